# baseline (speedup 1.0000x reference)
import functools

import jax
import jax.numpy as jnp
from jax import lax
from jax.experimental import pallas as pl
from jax.experimental.pallas import tpu as pltpu
from jax.experimental.pallas import tpu_sc as plsc

VOCAB = 1000000
DIM = 32
B = 4096
L = 200

NC = 2
NS = 16
NW = NC * NS
SEQ_W = B // NW
CH = 8
NCH = SEQ_W // CH
ROWS = CH * L
NBUF = 2


def _body(tok_hbm, table_hbm, out_hbm, idx_v, *bufs_sems):
    bufs = bufs_sems[:NBUF]
    gsems = bufs_sems[NBUF:2 * NBUF]
    wsems = bufs_sems[2 * NBUF:]
    wid = lax.axis_index("s") * NC + lax.axis_index("c")
    base = wid * SEQ_W * L
    pltpu.sync_copy(tok_hbm.at[wid], idx_v)

    def gather(c, b):
        pltpu.async_copy(table_hbm.at[idx_v.at[c]], bufs[b], gsems[b])

    def write(c, b):
        pltpu.async_copy(bufs[b],
                         out_hbm.at[pl.ds(base + c * ROWS, ROWS)], wsems[b])

    for b in range(NBUF):
        gather(b, b)

    @pl.loop(0, NCH, step=NBUF)
    def _grp(g):
        for b in range(NBUF):
            c = g + b
            pltpu.make_async_copy(table_hbm.at[idx_v.at[c]], bufs[b],
                                  gsems[b]).wait()
            write(c, b)

            @pl.when(c + NBUF < NCH)
            def _():
                pltpu.make_async_copy(bufs[b],
                                      out_hbm.at[pl.ds(0, ROWS)],
                                      wsems[b]).wait()
                gather(c + NBUF, b)

    for b in range(NBUF):
        pltpu.make_async_copy(bufs[b],
                              out_hbm.at[pl.ds(0, ROWS)],
                              wsems[b]).wait()


@functools.partial(jax.jit, static_argnames=())
def kernel(tokens, word_weight):
    grid_kernel = pl.kernel(
        _body,
        out_type=jax.ShapeDtypeStruct((B * L, DIM), jnp.float32),
        mesh=plsc.VectorSubcoreMesh(core_axis_name="c", subcore_axis_name="s"),
        scratch_types=(
            [pltpu.VMEM((NCH, ROWS), jnp.int32)]
            + [pltpu.VMEM((ROWS, DIM), jnp.float32)] * NBUF
            + [pltpu.SemaphoreType.DMA] * (2 * NBUF)
        ),
        compiler_params=pltpu.CompilerParams(use_tc_tiling_on_sc=False),
    )
    flat = grid_kernel(
        tokens.astype(jnp.int32).reshape(NW, NCH, ROWS),
        word_weight,
    )
    return flat.reshape(B, L, DIM)


# speedup vs baseline: 1.0011x; 1.0011x over previous
import functools

import jax
import jax.numpy as jnp
from jax import lax
from jax.experimental import pallas as pl
from jax.experimental.pallas import tpu as pltpu
from jax.experimental.pallas import tpu_sc as plsc

VOCAB = 1000000
DIM = 32
B = 4096
L = 200

NC = 2
NS = 16
NW = NC * NS
SEQ_W = B // NW
CH = 2
NCH = SEQ_W // CH
ROWS = CH * L
NBUF = 4


def _body(tok_hbm, table_hbm, out_hbm, idx_v, *bufs_sems):
    bufs = bufs_sems[:NBUF]
    gsems = bufs_sems[NBUF:2 * NBUF]
    wsems = bufs_sems[2 * NBUF:]
    wid = lax.axis_index("s") * NC + lax.axis_index("c")
    base = wid * SEQ_W * L
    pltpu.sync_copy(tok_hbm.at[wid], idx_v)

    def gather(c, b):
        pltpu.async_copy(table_hbm.at[idx_v.at[c]], bufs[b], gsems[b])

    def write(c, b):
        pltpu.async_copy(bufs[b],
                         out_hbm.at[pl.ds(base + c * ROWS, ROWS)], wsems[b])

    for b in range(NBUF):
        gather(b, b)

    @pl.loop(0, NCH, step=NBUF)
    def _grp(g):
        for b in range(NBUF):
            c = g + b
            pltpu.make_async_copy(table_hbm.at[idx_v.at[c]], bufs[b],
                                  gsems[b]).wait()
            write(c, b)

            @pl.when(c + NBUF < NCH)
            def _():
                pltpu.make_async_copy(bufs[b],
                                      out_hbm.at[pl.ds(0, ROWS)],
                                      wsems[b]).wait()
                gather(c + NBUF, b)

    for b in range(NBUF):
        pltpu.make_async_copy(bufs[b],
                              out_hbm.at[pl.ds(0, ROWS)],
                              wsems[b]).wait()


@functools.partial(jax.jit, static_argnames=())
def kernel(tokens, word_weight):
    grid_kernel = pl.kernel(
        _body,
        out_type=jax.ShapeDtypeStruct((B * L, DIM), jnp.float32),
        mesh=plsc.VectorSubcoreMesh(core_axis_name="c", subcore_axis_name="s"),
        scratch_types=(
            [pltpu.VMEM((NCH, ROWS), jnp.int32)]
            + [pltpu.VMEM((ROWS, DIM), jnp.float32)] * NBUF
            + [pltpu.SemaphoreType.DMA] * (2 * NBUF)
        ),
        compiler_params=pltpu.CompilerParams(use_tc_tiling_on_sc=False),
    )
    flat = grid_kernel(
        tokens.astype(jnp.int32).reshape(NW, NCH, ROWS),
        word_weight,
    )
    return flat.reshape(B, L, DIM)
